# Initial kernel scaffold; baseline (speedup 1.0000x reference)
import jax
import jax.numpy as jnp
from jax.experimental import pallas as pl


def kernel(x, edge_index, W_msg, W_upd, w_actor1, w_actor2):
    raise NotImplementedError("write your pallas kernel here")


# speedup vs baseline: 6.4820x; 6.4820x over previous
import functools

import jax
import jax.numpy as jnp
from jax import lax
from jax.experimental import pallas as pl
from jax.experimental.pallas import tpu as pltpu
from jax.experimental.pallas import tpu_sc as plsc

_N = 10000
_E = 320000
_D = 128
_H = 128
_A = 64

_NC = 2
_NS = 16
_TILES = _NC * _NS
_EPT = _E // _TILES
_CH = 128
_NFULL = _EPT // _CH
_TAIL = _EPT - _NFULL * _CH
_NP = 10240
_RPT = _NP // _NS
_ZR = 32


def _tc_pre_body(x_ref, wmsg_ref, wux_ref, y_ref, xu_ref):
    x = x_ref[...]
    y_ref[...] = jnp.maximum(jnp.dot(x, wmsg_ref[...]), 0.0)
    xu_ref[...] = jnp.dot(x, wux_ref[...])


_tc_pre = pl.pallas_call(
    _tc_pre_body,
    out_shape=(
        jax.ShapeDtypeStruct((_NP, _H), jnp.float32),
        jax.ShapeDtypeStruct((_NP, _H), jnp.float32),
    ),
)


_sc_mesh = plsc.VectorSubcoreMesh(core_axis_name="c", subcore_axis_name="s")


@functools.partial(
    pl.kernel,
    mesh=_sc_mesh,
    out_type=jax.ShapeDtypeStruct((_NC, _NP, _H), jnp.float32),
    scratch_types=[
        pltpu.VMEM((_CH,), jnp.int32),
        pltpu.VMEM((_CH,), jnp.int32),
        pltpu.VMEM((_CH, _H), jnp.float32),
        pltpu.VMEM((_TAIL,), jnp.int32),
        pltpu.VMEM((_TAIL,), jnp.int32),
        pltpu.VMEM((_TAIL, _H), jnp.float32),
        pltpu.VMEM((_ZR, _H), jnp.float32),
        pltpu.VMEM_SHARED((_NP, _H), jnp.float32),
        pltpu.SemaphoreType.DMA,
    ],
)
def _sc_agg(y_hbm, src_hbm, dst_hbm, out_hbm,
            src_v, dst_v, rows_v, srct_v, dstt_v, rowst_v,
            zbuf, agg_sh, sem):
    c = lax.axis_index("c")
    s = lax.axis_index("s")

    zero16 = jnp.zeros((16,), jnp.float32)
    for r in range(_ZR):
        for col in range(_H // 16):
            zbuf[r, pl.ds(col * 16, 16)] = zero16
    row0 = s * _RPT
    for j in range(_RPT // _ZR):
        pltpu.sync_copy(zbuf, agg_sh.at[pl.ds(row0 + j * _ZR, _ZR)])
    plsc.subcore_barrier()

    base_edges = c * (_E // _NC) + s * _EPT

    def body(i, carry):
        eb = pl.multiple_of(base_edges + i * _CH, 8)
        pltpu.sync_copy(src_hbm.at[pl.ds(eb, _CH)], src_v)
        pltpu.sync_copy(dst_hbm.at[pl.ds(eb, _CH)], dst_v)
        pltpu.async_copy(y_hbm.at[src_v], rows_v, sem).wait()
        pltpu.sync_copy(rows_v, agg_sh.at[dst_v], add=True)
        return carry

    lax.fori_loop(0, _NFULL, body, 0)

    eb = pl.multiple_of(base_edges + _NFULL * _CH, 8)
    pltpu.sync_copy(src_hbm.at[pl.ds(eb, _TAIL)], srct_v)
    pltpu.sync_copy(dst_hbm.at[pl.ds(eb, _TAIL)], dstt_v)
    pltpu.async_copy(y_hbm.at[srct_v], rowst_v, sem).wait()
    pltpu.sync_copy(rowst_v, agg_sh.at[dstt_v], add=True)

    plsc.subcore_barrier()
    pltpu.sync_copy(agg_sh.at[pl.ds(row0, _RPT)],
                    out_hbm.at[c, pl.ds(row0, _RPT)])


def _tc_post_body(xu_ref, aggs_ref, wua_ref, w1_ref, w2t_ref, out_ref):
    agg = aggs_ref[0, : _N] + aggs_ref[1, : _N]
    nf = jnp.maximum(xu_ref[: _N] + jnp.dot(agg, wua_ref[...]), 0.0)
    h1 = jnp.maximum(jnp.dot(nf, w1_ref[...]), 0.0)
    logits = jnp.sum(h1 * w2t_ref[...], axis=1, keepdims=True)
    m = jnp.max(logits)
    e = jnp.exp(logits - m)
    out_ref[...] = e / jnp.sum(e)


_tc_post = pl.pallas_call(
    _tc_post_body,
    out_shape=jax.ShapeDtypeStruct((_N, 1), jnp.float32),
)


def kernel(x, edge_index, W_msg, W_upd, w_actor1, w_actor2):
    src = edge_index[0]
    dst = edge_index[1]
    xp = jnp.concatenate(
        [x, jnp.zeros((_NP - _N, _D), jnp.float32)], axis=0)
    y, xu = _tc_pre(xp, W_msg, W_upd[:_D])
    aggs = _sc_agg(y, src, dst)
    probs = _tc_post(xu, aggs, W_upd[_D:], w_actor1, w_actor2.T)
    return probs[:, 0]
